# Initial kernel scaffold; baseline (speedup 1.0000x reference)
import jax
import jax.numpy as jnp
from jax.experimental import pallas as pl


def kernel(scores, times_sampled):
    raise NotImplementedError("write your pallas kernel here")


# speedup vs baseline: 5.9542x; 5.9542x over previous
import jax
import jax.numpy as jnp
from jax.experimental import pallas as pl
from jax.experimental.pallas import tpu as pltpu

_NEG = -1e30
_B = 32
_N = 4096
_K = 128
_BLK = 128
_NBLK = _N // _BLK
_INT_MIN = -2147483648


def _lse2(a, b):
    m = jnp.maximum(a, b)
    return m + jnp.log1p(jnp.exp(-jnp.abs(a - b)))


def _dp_topk_body(flat3_ref, pert_ref, samp_ref, marg3_ref,
                  lp_ref, lq_ref, chk_ref, fblk_ref, l_ref):
    x3 = flat3_ref[...]
    lp3 = jnp.minimum(x3, 0.0) - jnp.log1p(jnp.exp(-jnp.abs(x3)))
    lp_ref[...] = lp3
    lq_ref[...] = lp3 - x3

    ids = jax.lax.broadcasted_iota(jnp.int32, (_B, _BLK), 1)
    neg_col = jnp.full((_B, 1), _NEG, dtype=jnp.float32)
    f_init = jnp.where(ids == 0, 0.0, _NEG)
    br_init = jnp.where(ids == _BLK - 1, 0.0, _NEG)

    def _col(tile, j):
        return jnp.sum(jnp.where(ids == j, tile, 0.0), axis=1, keepdims=True)

    def _step(lpt, lqt, j, c):
        lpi = _col(lpt, j)
        lqi = _col(lqt, j)
        shifted = jnp.concatenate([neg_col, c[:, :-1]], axis=1)
        return _lse2(c + lqi, shifted + lpi)

    def fwd_block(blk, carry):
        chk_ref[pl.ds(blk, 1)] = carry[None]
        lpt = lp_ref[blk]
        lqt = lq_ref[blk]

        def step(j, c):
            return _step(lpt, lqt, j, c)

        return jax.lax.fori_loop(0, _BLK, step, carry)

    jax.lax.fori_loop(0, _NBLK, fwd_block, f_init)

    def bwd_block(t, br):
        blk = _NBLK - 1 - t
        lpt = lp_ref[blk]
        lqt = lq_ref[blk]

        def refill(j, c):
            fblk_ref[pl.ds(j, 1)] = c[None]
            return _step(lpt, lqt, j, c)

        jax.lax.fori_loop(0, _BLK, refill, chk_ref[blk])

        def step(tt, carry2):
            br, ltile = carry2
            j = _BLK - 1 - tt
            f = fblk_ref[j]
            s = f + br
            m = jnp.max(s, axis=1, keepdims=True)
            lsum = m + jnp.log(jnp.sum(jnp.exp(s - m), axis=1, keepdims=True))
            lpi = _col(lpt, j)
            lqi = _col(lqt, j)
            ltile = jnp.where(ids == j, lpi + lsum, ltile)
            shifted = jnp.concatenate([br[:, 1:], neg_col], axis=1)
            br = _lse2(br + lqi, shifted + lpi)
            return br, ltile

        br, ltile = jax.lax.fori_loop(
            0, _BLK, step, (br, jnp.zeros((_B, _BLK), jnp.float32)))
        l_ref[pl.ds(blk, 1)] = ltile[None]
        return br

    jax.lax.fori_loop(0, _NBLK, bwd_block, br_init)

    l = l_ref[...]
    lm = jnp.max(l, axis=(0, 2), keepdims=True)
    e = jnp.exp(l - lm)
    marg = float(_K) * e / jnp.sum(e, axis=(0, 2), keepdims=True)
    marg3_ref[...] = jnp.clip(marg, 0.0, 1.0)

    bits = jax.lax.bitcast_convert_type(pert_ref[...], jnp.int32)
    key = bits ^ (jnp.int32(0x7FFFFFFF) & (bits >> 31))
    tu = jnp.zeros((_B, 1), jnp.int32)
    for bit in range(31, -1, -1):
        c = 1 << bit
        c = c - (1 << 32) if c >= (1 << 31) else c
        cand = tu | jnp.int32(c)
        thr = cand ^ jnp.int32(_INT_MIN)
        cnt = jnp.sum((key >= thr).astype(jnp.int32), axis=1, keepdims=True)
        tu = jnp.where(cnt >= _K, cand, tu)
    tkey = tu ^ jnp.int32(_INT_MIN)
    gt = key > tkey
    eq = key == tkey
    need = _K - jnp.sum(gt.astype(jnp.int32), axis=1, keepdims=True)
    r = eq.astype(jnp.int32)
    sh = 1
    while sh < _N:
        r = r + jnp.concatenate(
            [jnp.zeros((_B, sh), jnp.int32), r[:, :-sh]], axis=1)
        sh *= 2
    mask = gt | (eq & (r <= need))
    samp_ref[...] = mask.astype(jnp.float32)


def _run(flat3, pert):
    return pl.pallas_call(
        _dp_topk_body,
        out_shape=(
            jax.ShapeDtypeStruct((_B, _N), jnp.float32),
            jax.ShapeDtypeStruct((_NBLK, _B, _BLK), jnp.float32),
        ),
        scratch_shapes=[
            pltpu.VMEM((_NBLK, _B, _BLK), jnp.float32),
            pltpu.VMEM((_NBLK, _B, _BLK), jnp.float32),
            pltpu.VMEM((_NBLK, _B, _BLK), jnp.float32),
            pltpu.VMEM((_BLK, _B, _BLK), jnp.float32),
            pltpu.VMEM((_NBLK, _B, _BLK), jnp.float32),
        ],
    )(flat3, pert)


def kernel(scores, times_sampled):
    bsz, nmax, _, ens = scores.shape
    flat = jnp.transpose(scores, (0, 3, 1, 2)).reshape(bsz * ens, nmax * nmax)
    g = jax.random.gumbel(jax.random.key(42), flat.shape, flat.dtype)
    flat3 = flat.reshape(_B, _NBLK, _BLK).transpose(1, 0, 2)
    samples, marg3 = _run(flat3, flat + g)
    marg = marg3.transpose(1, 0, 2).reshape(_B, _N)
    new_mask = jnp.transpose(
        samples.reshape(1, bsz, ens, nmax, nmax), (0, 1, 3, 4, 2))
    new_marginals = jnp.transpose(
        marg.reshape(bsz, ens, nmax, nmax), (0, 2, 3, 1))
    return new_mask, new_marginals


# speedup vs baseline: 9.9919x; 1.6781x over previous
import jax
import jax.numpy as jnp
from jax.experimental import pallas as pl
from jax.experimental.pallas import tpu as pltpu

_NEG = -1e30
_B = 32
_N = 4096
_K = 128
_BLK = 128
_NBLK = _N // _BLK
_INT_MIN = -2147483648

_IDS = None


def _lse2(a, b):
    m = jnp.maximum(a, b)
    return m + jnp.log1p(jnp.exp(-jnp.abs(a - b)))


def _logpq(x):
    lp = jnp.minimum(x, 0.0) - jnp.log1p(jnp.exp(-jnp.abs(x)))
    return lp, lp - x


def _col(ids, tile, j):
    return jnp.sum(jnp.where(ids == j, tile, 0.0), axis=1, keepdims=True)


def _fwd_body(flat_ref, fout_ref, carry_ref):
    b = pl.program_id(0)
    ids = jax.lax.broadcasted_iota(jnp.int32, (_B, _BLK), 1)
    neg_col = jnp.full((_B, 1), _NEG, dtype=jnp.float32)

    @pl.when(b == 0)
    def _():
        carry_ref[...] = jnp.where(ids == 0, 0.0, _NEG)

    lpt, lqt = _logpq(flat_ref[0])

    def step(j, c):
        fout_ref[0, pl.ds(j, 1)] = c[None]
        lpi = _col(ids, lpt, j)
        lqi = _col(ids, lqt, j)
        shifted = jnp.concatenate([neg_col, c[:, :-1]], axis=1)
        return _lse2(c + lqi, shifted + lpi)

    carry_ref[...] = jax.lax.fori_loop(0, _BLK, step, carry_ref[...])


def _bwd_body(flat_ref, fpref_ref, l_ref, br_ref, bblk_ref):
    g = pl.program_id(0)
    ids = jax.lax.broadcasted_iota(jnp.int32, (_B, _BLK), 1)
    neg_col = jnp.full((_B, 1), _NEG, dtype=jnp.float32)

    @pl.when(g == 0)
    def _():
        br_ref[...] = jnp.where(ids == _BLK - 1, 0.0, _NEG)

    lpt, lqt = _logpq(flat_ref[0])

    def step(tt, br):
        j = _BLK - 1 - tt
        bblk_ref[pl.ds(j, 1)] = br[None]
        lpi = _col(ids, lpt, j)
        lqi = _col(ids, lqt, j)
        shifted = jnp.concatenate([br[:, 1:], neg_col], axis=1)
        return _lse2(br + lqi, shifted + lpi)

    br_ref[...] = jax.lax.fori_loop(0, _BLK, step, br_ref[...])

    s = fpref_ref[0] + bblk_ref[...]
    m = jnp.max(s, axis=2, keepdims=True)
    lsum = m + jnp.log(jnp.sum(jnp.exp(s - m), axis=2, keepdims=True))
    l_ref[0] = lpt + jnp.transpose(lsum[:, :, 0])


def _fin_body(l_ref, pert_ref, samp_ref, marg3_ref):
    l = l_ref[...]
    lm = jnp.max(l, axis=(0, 2), keepdims=True)
    e = jnp.exp(l - lm)
    marg = float(_K) * e / jnp.sum(e, axis=(0, 2), keepdims=True)
    marg3_ref[...] = jnp.clip(marg, 0.0, 1.0)

    bits = jax.lax.bitcast_convert_type(pert_ref[...], jnp.int32)
    key = bits ^ (jnp.int32(0x7FFFFFFF) & (bits >> 31))
    tu = jnp.zeros((_B, 1), jnp.int32)
    for bit in range(31, -1, -1):
        c = 1 << bit
        c = c - (1 << 32) if c >= (1 << 31) else c
        cand = tu | jnp.int32(c)
        thr = cand ^ jnp.int32(_INT_MIN)
        cnt = jnp.sum((key >= thr).astype(jnp.int32), axis=1, keepdims=True)
        tu = jnp.where(cnt >= _K, cand, tu)
    tkey = tu ^ jnp.int32(_INT_MIN)
    gt = key > tkey
    eq = key == tkey
    need = _K - jnp.sum(gt.astype(jnp.int32), axis=1, keepdims=True)
    r = eq.astype(jnp.int32)
    sh = 1
    while sh < _N:
        r = r + jnp.concatenate(
            [jnp.zeros((_B, sh), jnp.int32), r[:, :-sh]], axis=1)
        sh *= 2
    mask = gt | (eq & (r <= need))
    samp_ref[...] = mask.astype(jnp.float32)


def _run(flat3, pert):
    fpref = pl.pallas_call(
        _fwd_body,
        grid=(_NBLK,),
        in_specs=[pl.BlockSpec((1, _B, _BLK), lambda b: (b, 0, 0))],
        out_specs=pl.BlockSpec((1, _BLK, _B, _BLK), lambda b: (b, 0, 0, 0)),
        out_shape=jax.ShapeDtypeStruct((_NBLK, _BLK, _B, _BLK), jnp.float32),
        scratch_shapes=[pltpu.VMEM((_B, _BLK), jnp.float32)],
    )(flat3)

    l3 = pl.pallas_call(
        _bwd_body,
        grid=(_NBLK,),
        in_specs=[
            pl.BlockSpec((1, _B, _BLK), lambda g: (_NBLK - 1 - g, 0, 0)),
            pl.BlockSpec((1, _BLK, _B, _BLK),
                         lambda g: (_NBLK - 1 - g, 0, 0, 0)),
        ],
        out_specs=pl.BlockSpec((1, _B, _BLK), lambda g: (_NBLK - 1 - g, 0, 0)),
        out_shape=jax.ShapeDtypeStruct((_NBLK, _B, _BLK), jnp.float32),
        scratch_shapes=[
            pltpu.VMEM((_B, _BLK), jnp.float32),
            pltpu.VMEM((_BLK, _B, _BLK), jnp.float32),
        ],
    )(flat3, fpref)

    return pl.pallas_call(
        _fin_body,
        out_shape=(
            jax.ShapeDtypeStruct((_B, _N), jnp.float32),
            jax.ShapeDtypeStruct((_NBLK, _B, _BLK), jnp.float32),
        ),
    )(l3, pert)


def kernel(scores, times_sampled):
    bsz, nmax, _, ens = scores.shape
    flat = jnp.transpose(scores, (0, 3, 1, 2)).reshape(bsz * ens, nmax * nmax)
    g = jax.random.gumbel(jax.random.key(42), flat.shape, flat.dtype)
    flat3 = flat.reshape(_B, _NBLK, _BLK).transpose(1, 0, 2)
    samples, marg3 = _run(flat3, flat + g)
    marg = marg3.transpose(1, 0, 2).reshape(_B, _N)
    new_mask = jnp.transpose(
        samples.reshape(1, bsz, ens, nmax, nmax), (0, 1, 3, 4, 2))
    new_marginals = jnp.transpose(
        marg.reshape(bsz, ens, nmax, nmax), (0, 2, 3, 1))
    return new_mask, new_marginals


# speedup vs baseline: 12.0019x; 1.2012x over previous
import jax
import jax.numpy as jnp
from jax.experimental import pallas as pl
from jax.experimental.pallas import tpu as pltpu

_NEG = -1e30
_B = 32
_N = 4096
_K = 128
_BLK = 128
_NBLK = _N // _BLK
_R = 4
_G = _BLK // _R
_INT_MIN = -2147483648


def _lse2(a, b):
    m = jnp.maximum(a, b)
    return m + jnp.log1p(jnp.exp(-jnp.abs(a - b)))


def _logpq(x):
    lp = jnp.minimum(x, 0.0) - jnp.log1p(jnp.exp(-jnp.abs(x)))
    return lp, lp - x


def _group_taps(lp4, lq4):
    z = jnp.zeros_like(lp4[0])
    t = [z, jnp.full_like(z, _NEG), jnp.full_like(z, _NEG),
         jnp.full_like(z, _NEG), jnp.full_like(z, _NEG)]
    for s in range(_R):
        lps, lqs = lp4[s], lq4[s]
        for tau in range(_R, 0, -1):
            t[tau] = _lse2(t[tau] + lqs, t[tau - 1] + lps)
        t[0] = t[0] + lqs
    return t


def _loo_taps(lp4, lq4):
    outs = []
    z = jnp.zeros_like(lp4[0])
    for s in range(_R):
        e = [z, jnp.full_like(z, _NEG), jnp.full_like(z, _NEG),
             jnp.full_like(z, _NEG)]
        for s2 in range(_R):
            if s2 == s:
                continue
            lps, lqs = lp4[s2], lq4[s2]
            for tau in range(_R - 1, 0, -1):
                e[tau] = _lse2(e[tau] + lqs, e[tau - 1] + lps)
            e[0] = e[0] + lqs
        outs.append(e)
    return outs


def _col(ids, tile, j):
    return jnp.sum(jnp.where(ids == j, tile, 0.0), axis=1, keepdims=True)


def _fused_step(c, taps, reverse):
    neg_col = jnp.full((c.shape[0], 1), _NEG, dtype=c.dtype)
    xs = [c + taps[0]]
    sh = c
    for tau in range(1, _R + 1):
        if reverse:
            sh = jnp.concatenate([sh[:, 1:], neg_col], axis=1)
        else:
            sh = jnp.concatenate([neg_col, sh[:, :-1]], axis=1)
        xs.append(sh + taps[tau])
    m = xs[0]
    for x in xs[1:]:
        m = jnp.maximum(m, x)
    acc = jnp.exp(xs[0] - m)
    for x in xs[1:]:
        acc = acc + jnp.exp(x - m)
    return m + jnp.log(acc)


def _fwd_body(flat4_ref, fout_ref, carry_ref):
    b = pl.program_id(0)
    idsk = jax.lax.broadcasted_iota(jnp.int32, (_B, _BLK), 1)
    idsg = jax.lax.broadcasted_iota(jnp.int32, (_B, _G), 1)

    @pl.when(b == 0)
    def _():
        carry_ref[...] = jnp.where(idsk == 0, 0.0, _NEG)

    lp4, lq4 = _logpq(flat4_ref[0])
    t = _group_taps(lp4, lq4)

    def step(g, c):
        fout_ref[0, pl.ds(g, 1)] = c[None]
        taps = [_col(idsg, t[tau], g) for tau in range(_R + 1)]
        return _fused_step(c, taps, reverse=False)

    carry_ref[...] = jax.lax.fori_loop(0, _G, step, carry_ref[...])


def _bwd_body(flat4_ref, fpref_ref, l_ref, br_ref, bblk_ref):
    gidx = pl.program_id(0)
    idsk = jax.lax.broadcasted_iota(jnp.int32, (_B, _BLK), 1)
    idsg = jax.lax.broadcasted_iota(jnp.int32, (_B, _G), 1)

    @pl.when(gidx == 0)
    def _():
        br_ref[...] = jnp.where(idsk == _BLK - 1, 0.0, _NEG)

    lp4, lq4 = _logpq(flat4_ref[0])
    t = _group_taps(lp4, lq4)

    def step(tt, br):
        g = _G - 1 - tt
        bblk_ref[pl.ds(g, 1)] = br[None]
        taps = [_col(idsg, t[tau], g) for tau in range(_R + 1)]
        return _fused_step(br, taps, reverse=True)

    br_ref[...] = jax.lax.fori_loop(0, _G, step, br_ref[...])

    fblk = fpref_ref[0]
    brblk = bblk_ref[...]
    dts = []
    for tau in range(_R):
        if tau:
            sh = jnp.concatenate(
                [brblk[:, :, tau:],
                 jnp.full((_G, _B, tau), _NEG, jnp.float32)], axis=2)
        else:
            sh = brblk
        s = fblk + sh
        m = jnp.max(s, axis=2, keepdims=True)
        d = m + jnp.log(jnp.sum(jnp.exp(s - m), axis=2, keepdims=True))
        dts.append(jnp.transpose(d[:, :, 0]))

    loo = _loo_taps(lp4, lq4)
    ls = []
    for s in range(_R):
        xs = [loo[s][tau] + dts[tau] for tau in range(_R)]
        m = xs[0]
        for x in xs[1:]:
            m = jnp.maximum(m, x)
        acc = jnp.exp(xs[0] - m)
        for x in xs[1:]:
            acc = acc + jnp.exp(x - m)
        ls.append(lp4[s] + m + jnp.log(acc))
    l_ref[0] = jnp.stack(ls, axis=0)


def _fin_body(l_ref, pert_ref, samp_ref, marg4_ref):
    l = l_ref[...]
    lm = jnp.max(l, axis=(0, 1, 3), keepdims=True)
    e = jnp.exp(l - lm)
    marg = float(_K) * e / jnp.sum(e, axis=(0, 1, 3), keepdims=True)
    marg4_ref[...] = jnp.clip(marg, 0.0, 1.0)

    bits = jax.lax.bitcast_convert_type(pert_ref[...], jnp.int32)
    key = bits ^ (jnp.int32(0x7FFFFFFF) & (bits >> 31))
    tu = jnp.zeros((_B, 1), jnp.int32)
    for bit in range(31, -1, -1):
        c = 1 << bit
        c = c - (1 << 32) if c >= (1 << 31) else c
        cand = tu | jnp.int32(c)
        thr = cand ^ jnp.int32(_INT_MIN)
        cnt = jnp.sum((key >= thr).astype(jnp.int32), axis=1, keepdims=True)
        tu = jnp.where(cnt >= _K, cand, tu)
    tkey = tu ^ jnp.int32(_INT_MIN)
    gt = key > tkey
    eq = key == tkey
    need = _K - jnp.sum(gt.astype(jnp.int32), axis=1, keepdims=True)
    r = eq.astype(jnp.int32)
    sh = 1
    while sh < _N:
        r = r + jnp.concatenate(
            [jnp.zeros((_B, sh), jnp.int32), r[:, :-sh]], axis=1)
        sh *= 2
    mask = gt | (eq & (r <= need))
    samp_ref[...] = mask.astype(jnp.float32)


def _run(flat4, pert):
    fpref = pl.pallas_call(
        _fwd_body,
        grid=(_NBLK,),
        in_specs=[pl.BlockSpec((1, _R, _B, _G), lambda b: (b, 0, 0, 0))],
        out_specs=pl.BlockSpec((1, _G, _B, _BLK), lambda b: (b, 0, 0, 0)),
        out_shape=jax.ShapeDtypeStruct((_NBLK, _G, _B, _BLK), jnp.float32),
        scratch_shapes=[pltpu.VMEM((_B, _BLK), jnp.float32)],
    )(flat4)

    l4 = pl.pallas_call(
        _bwd_body,
        grid=(_NBLK,),
        in_specs=[
            pl.BlockSpec((1, _R, _B, _G), lambda g: (_NBLK - 1 - g, 0, 0, 0)),
            pl.BlockSpec((1, _G, _B, _BLK),
                         lambda g: (_NBLK - 1 - g, 0, 0, 0)),
        ],
        out_specs=pl.BlockSpec((1, _R, _B, _G),
                               lambda g: (_NBLK - 1 - g, 0, 0, 0)),
        out_shape=jax.ShapeDtypeStruct((_NBLK, _R, _B, _G), jnp.float32),
        scratch_shapes=[
            pltpu.VMEM((_B, _BLK), jnp.float32),
            pltpu.VMEM((_G, _B, _BLK), jnp.float32),
        ],
    )(flat4, fpref)

    return pl.pallas_call(
        _fin_body,
        out_shape=(
            jax.ShapeDtypeStruct((_B, _N), jnp.float32),
            jax.ShapeDtypeStruct((_NBLK, _R, _B, _G), jnp.float32),
        ),
    )(l4, pert)


def kernel(scores, times_sampled):
    bsz, nmax, _, ens = scores.shape
    flat = jnp.transpose(scores, (0, 3, 1, 2)).reshape(bsz * ens, nmax * nmax)
    g = jax.random.gumbel(jax.random.key(42), flat.shape, flat.dtype)
    flat4 = flat.reshape(_B, _NBLK, _G, _R).transpose(1, 3, 0, 2)
    samples, marg4 = _run(flat4, flat + g)
    marg = marg4.transpose(2, 0, 3, 1).reshape(_B, _N)
    new_mask = jnp.transpose(
        samples.reshape(1, bsz, ens, nmax, nmax), (0, 1, 3, 4, 2))
    new_marginals = jnp.transpose(
        marg.reshape(bsz, ens, nmax, nmax), (0, 2, 3, 1))
    return new_mask, new_marginals


# speedup vs baseline: 13.8269x; 1.1521x over previous
import jax
import jax.numpy as jnp
from jax.experimental import pallas as pl
from jax.experimental.pallas import tpu as pltpu

_NEG = -1e30
_B = 32
_N = 4096
_K = 128
_BLK = 128
_NBLK = _N // _BLK
_R = 4
_G = _BLK // _R
_INT_MIN = -2147483648


def _lse2(a, b):
    m = jnp.maximum(a, b)
    return m + jnp.log1p(jnp.exp(-jnp.abs(a - b)))


def _logpq(x):
    lp = jnp.minimum(x, 0.0) - jnp.log1p(jnp.exp(-jnp.abs(x)))
    return lp, lp - x


def _group_taps(lp4, lq4):
    z = jnp.zeros_like(lp4[0])
    t = [z, jnp.full_like(z, _NEG), jnp.full_like(z, _NEG),
         jnp.full_like(z, _NEG), jnp.full_like(z, _NEG)]
    for s in range(_R):
        lps, lqs = lp4[s], lq4[s]
        for tau in range(_R, 0, -1):
            t[tau] = _lse2(t[tau] + lqs, t[tau - 1] + lps)
        t[0] = t[0] + lqs
    return t


def _loo_taps(lp4, lq4):
    outs = []
    z = jnp.zeros_like(lp4[0])
    for s in range(_R):
        e = [z, jnp.full_like(z, _NEG), jnp.full_like(z, _NEG),
             jnp.full_like(z, _NEG)]
        for s2 in range(_R):
            if s2 == s:
                continue
            lps, lqs = lp4[s2], lq4[s2]
            for tau in range(_R - 1, 0, -1):
                e[tau] = _lse2(e[tau] + lqs, e[tau - 1] + lps)
            e[0] = e[0] + lqs
        outs.append(e)
    return outs


def _col(tile, j):
    return tile[:, j:j + 1]


def _fused_step(c, taps, reverse):
    neg_col = jnp.full((c.shape[0], 1), _NEG, dtype=c.dtype)
    xs = [c + taps[0]]
    sh = c
    for tau in range(1, _R + 1):
        if reverse:
            sh = jnp.concatenate([sh[:, 1:], neg_col], axis=1)
        else:
            sh = jnp.concatenate([neg_col, sh[:, :-1]], axis=1)
        xs.append(sh + taps[tau])
    m = xs[0]
    for x in xs[1:]:
        m = jnp.maximum(m, x)
    acc = jnp.exp(xs[0] - m)
    for x in xs[1:]:
        acc = acc + jnp.exp(x - m)
    return m + jnp.log(acc)


def _fwd_body(flat4_ref, fout_ref, carry_ref):
    b = pl.program_id(0)
    idsk = jax.lax.broadcasted_iota(jnp.int32, (_B, _BLK), 1)

    @pl.when(b == 0)
    def _():
        carry_ref[...] = jnp.where(idsk == 0, 0.0, _NEG)

    lp4, lq4 = _logpq(flat4_ref[0])
    t = _group_taps(lp4, lq4)

    c = carry_ref[...]
    for g in range(_G):
        fout_ref[0, g] = c
        taps = [_col(t[tau], g) for tau in range(_R + 1)]
        c = _fused_step(c, taps, reverse=False)
    carry_ref[...] = c


def _bwd_body(flat4_ref, fpref_ref, l_ref, br_ref, bblk_ref):
    gidx = pl.program_id(0)
    idsk = jax.lax.broadcasted_iota(jnp.int32, (_B, _BLK), 1)

    @pl.when(gidx == 0)
    def _():
        br_ref[...] = jnp.where(idsk == _BLK - 1, 0.0, _NEG)

    lp4, lq4 = _logpq(flat4_ref[0])
    t = _group_taps(lp4, lq4)

    br = br_ref[...]
    for g in range(_G - 1, -1, -1):
        bblk_ref[g] = br
        taps = [_col(t[tau], g) for tau in range(_R + 1)]
        br = _fused_step(br, taps, reverse=True)
    br_ref[...] = br

    fblk = fpref_ref[0]
    brblk = bblk_ref[...]
    dts = []
    for tau in range(_R):
        if tau:
            sh = jnp.concatenate(
                [brblk[:, :, tau:],
                 jnp.full((_G, _B, tau), _NEG, jnp.float32)], axis=2)
        else:
            sh = brblk
        s = fblk + sh
        m = jnp.max(s, axis=2, keepdims=True)
        d = m + jnp.log(jnp.sum(jnp.exp(s - m), axis=2, keepdims=True))
        dts.append(jnp.transpose(d[:, :, 0]))

    loo = _loo_taps(lp4, lq4)
    ls = []
    for s in range(_R):
        xs = [loo[s][tau] + dts[tau] for tau in range(_R)]
        m = xs[0]
        for x in xs[1:]:
            m = jnp.maximum(m, x)
        acc = jnp.exp(xs[0] - m)
        for x in xs[1:]:
            acc = acc + jnp.exp(x - m)
        ls.append(lp4[s] + m + jnp.log(acc))
    l_ref[0] = jnp.stack(ls, axis=0)


def _fin_body(l_ref, pert_ref, samp_ref, marg4_ref):
    l = l_ref[...]
    lm = jnp.max(l, axis=(0, 1, 3), keepdims=True)
    e = jnp.exp(l - lm)
    marg = float(_K) * e / jnp.sum(e, axis=(0, 1, 3), keepdims=True)
    marg4_ref[...] = jnp.clip(marg, 0.0, 1.0)

    bits = jax.lax.bitcast_convert_type(pert_ref[...], jnp.int32)
    key = bits ^ (jnp.int32(0x7FFFFFFF) & (bits >> 31))
    tu = jnp.zeros((_B, 1), jnp.int32)
    for bit in range(31, -1, -1):
        c = 1 << bit
        c = c - (1 << 32) if c >= (1 << 31) else c
        cand = tu | jnp.int32(c)
        thr = cand ^ jnp.int32(_INT_MIN)
        cnt = jnp.sum((key >= thr).astype(jnp.int32), axis=1, keepdims=True)
        tu = jnp.where(cnt >= _K, cand, tu)
    tkey = tu ^ jnp.int32(_INT_MIN)
    gt = key > tkey
    eq = key == tkey
    need = _K - jnp.sum(gt.astype(jnp.int32), axis=1, keepdims=True)
    r = eq.astype(jnp.int32)
    sh = 1
    while sh < _N:
        r = r + jnp.concatenate(
            [jnp.zeros((_B, sh), jnp.int32), r[:, :-sh]], axis=1)
        sh *= 2
    mask = gt | (eq & (r <= need))
    samp_ref[...] = mask.astype(jnp.float32)


def _run(flat4, pert):
    fpref = pl.pallas_call(
        _fwd_body,
        grid=(_NBLK,),
        in_specs=[pl.BlockSpec((1, _R, _B, _G), lambda b: (b, 0, 0, 0))],
        out_specs=pl.BlockSpec((1, _G, _B, _BLK), lambda b: (b, 0, 0, 0)),
        out_shape=jax.ShapeDtypeStruct((_NBLK, _G, _B, _BLK), jnp.float32),
        scratch_shapes=[pltpu.VMEM((_B, _BLK), jnp.float32)],
    )(flat4)

    l4 = pl.pallas_call(
        _bwd_body,
        grid=(_NBLK,),
        in_specs=[
            pl.BlockSpec((1, _R, _B, _G), lambda g: (_NBLK - 1 - g, 0, 0, 0)),
            pl.BlockSpec((1, _G, _B, _BLK),
                         lambda g: (_NBLK - 1 - g, 0, 0, 0)),
        ],
        out_specs=pl.BlockSpec((1, _R, _B, _G),
                               lambda g: (_NBLK - 1 - g, 0, 0, 0)),
        out_shape=jax.ShapeDtypeStruct((_NBLK, _R, _B, _G), jnp.float32),
        scratch_shapes=[
            pltpu.VMEM((_B, _BLK), jnp.float32),
            pltpu.VMEM((_G, _B, _BLK), jnp.float32),
        ],
    )(flat4, fpref)

    return pl.pallas_call(
        _fin_body,
        out_shape=(
            jax.ShapeDtypeStruct((_B, _N), jnp.float32),
            jax.ShapeDtypeStruct((_NBLK, _R, _B, _G), jnp.float32),
        ),
    )(l4, pert)


def kernel(scores, times_sampled):
    bsz, nmax, _, ens = scores.shape
    flat = jnp.transpose(scores, (0, 3, 1, 2)).reshape(bsz * ens, nmax * nmax)
    g = jax.random.gumbel(jax.random.key(42), flat.shape, flat.dtype)
    flat4 = flat.reshape(_B, _NBLK, _G, _R).transpose(1, 3, 0, 2)
    samples, marg4 = _run(flat4, flat + g)
    marg = marg4.transpose(2, 0, 3, 1).reshape(_B, _N)
    new_mask = jnp.transpose(
        samples.reshape(1, bsz, ens, nmax, nmax), (0, 1, 3, 4, 2))
    new_marginals = jnp.transpose(
        marg.reshape(bsz, ens, nmax, nmax), (0, 2, 3, 1))
    return new_mask, new_marginals


# speedup vs baseline: 21.9022x; 1.5840x over previous
import jax
import jax.numpy as jnp
from jax.experimental import pallas as pl
from jax.experimental.pallas import tpu as pltpu

_NEG = -1e30
_B = 32
_N = 4096
_K = 128
_BLK = 128
_NBLK = _N // _BLK
_R = 4
_G = _BLK // _R
_INT_MIN = -2147483648


def _lse2(a, b):
    m = jnp.maximum(a, b)
    return m + jnp.log1p(jnp.exp(-jnp.abs(a - b)))


def _logpq(x):
    lp = jnp.minimum(x, 0.0) - jnp.log1p(jnp.exp(-jnp.abs(x)))
    return lp, lp - x


def _group_taps(lp4, lq4):
    z = jnp.zeros_like(lp4[0])
    t = [z, jnp.full_like(z, _NEG), jnp.full_like(z, _NEG),
         jnp.full_like(z, _NEG), jnp.full_like(z, _NEG)]
    for s in range(_R):
        lps, lqs = lp4[s], lq4[s]
        for tau in range(_R, 0, -1):
            t[tau] = _lse2(t[tau] + lqs, t[tau - 1] + lps)
        t[0] = t[0] + lqs
    return t


def _loo_taps(lp4, lq4):
    outs = []
    z = jnp.zeros_like(lp4[0])
    for s in range(_R):
        e = [z, jnp.full_like(z, _NEG), jnp.full_like(z, _NEG),
             jnp.full_like(z, _NEG)]
        for s2 in range(_R):
            if s2 == s:
                continue
            lps, lqs = lp4[s2], lq4[s2]
            for tau in range(_R - 1, 0, -1):
                e[tau] = _lse2(e[tau] + lqs, e[tau - 1] + lps)
            e[0] = e[0] + lqs
        outs.append(e)
    return outs


def _col(tile, j):
    return tile[:, j:j + 1]


def _fused_step(c, taps, reverse):
    neg_col = jnp.full((c.shape[0], 1), _NEG, dtype=c.dtype)
    xs = [c + taps[0]]
    sh = c
    for tau in range(1, _R + 1):
        if reverse:
            sh = jnp.concatenate([sh[:, 1:], neg_col], axis=1)
        else:
            sh = jnp.concatenate([neg_col, sh[:, :-1]], axis=1)
        xs.append(sh + taps[tau])
    m = xs[0]
    for x in xs[1:]:
        m = jnp.maximum(m, x)
    acc = jnp.exp(xs[0] - m)
    for x in xs[1:]:
        acc = acc + jnp.exp(x - m)
    return m + jnp.log(acc)


def _dp_body(flatf_ref, flatb_ref, fout_ref, bout_ref, fc_ref, bc_ref):
    g = pl.program_id(0)
    idsk = jax.lax.broadcasted_iota(jnp.int32, (_B, _BLK), 1)

    @pl.when(g == 0)
    def _():
        fc_ref[...] = jnp.where(idsk == 0, 0.0, _NEG)
        bc_ref[...] = jnp.where(idsk == _BLK - 1, 0.0, _NEG)

    lp4f, lq4f = _logpq(flatf_ref[0])
    tf = _group_taps(lp4f, lq4f)
    lp4b, lq4b = _logpq(flatb_ref[0])
    tb = _group_taps(lp4b, lq4b)

    c = fc_ref[...]
    br = bc_ref[...]
    for gg in range(_G):
        fout_ref[0, gg] = c
        bout_ref[0, _G - 1 - gg] = br
        ftaps = [_col(tf[tau], gg) for tau in range(_R + 1)]
        btaps = [_col(tb[tau], _G - 1 - gg) for tau in range(_R + 1)]
        c = _fused_step(c, ftaps, reverse=False)
        br = _fused_step(br, btaps, reverse=True)
    fc_ref[...] = c
    bc_ref[...] = br


def _comb_body(flat4_ref, fpref_ref, bpref_ref, l_ref):
    lp4, lq4 = _logpq(flat4_ref[0])

    fblk = fpref_ref[0]
    brblk = bpref_ref[0]
    dts = []
    for tau in range(_R):
        if tau:
            sh = jnp.concatenate(
                [brblk[:, :, tau:],
                 jnp.full((_G, _B, tau), _NEG, jnp.float32)], axis=2)
        else:
            sh = brblk
        s = fblk + sh
        m = jnp.max(s, axis=2, keepdims=True)
        d = m + jnp.log(jnp.sum(jnp.exp(s - m), axis=2, keepdims=True))
        dts.append(jnp.transpose(d[:, :, 0]))

    loo = _loo_taps(lp4, lq4)
    ls = []
    for s in range(_R):
        xs = [loo[s][tau] + dts[tau] for tau in range(_R)]
        m = xs[0]
        for x in xs[1:]:
            m = jnp.maximum(m, x)
        acc = jnp.exp(xs[0] - m)
        for x in xs[1:]:
            acc = acc + jnp.exp(x - m)
        ls.append(lp4[s] + m + jnp.log(acc))
    l_ref[0] = jnp.stack(ls, axis=0)


def _fin_body(l_ref, pert_ref, samp_ref, marg4_ref):
    l = l_ref[...]
    lm = jnp.max(l, axis=(0, 1, 3), keepdims=True)
    e = jnp.exp(l - lm)
    marg = float(_K) * e / jnp.sum(e, axis=(0, 1, 3), keepdims=True)
    marg4_ref[...] = jnp.clip(marg, 0.0, 1.0)

    bits = jax.lax.bitcast_convert_type(pert_ref[...], jnp.int32)
    key = bits ^ (jnp.int32(0x7FFFFFFF) & (bits >> 31))
    tu = jnp.zeros((_B, 1), jnp.int32)
    for bit in range(31, -1, -1):
        c = 1 << bit
        c = c - (1 << 32) if c >= (1 << 31) else c
        cand = tu | jnp.int32(c)
        thr = cand ^ jnp.int32(_INT_MIN)
        cnt = jnp.sum((key >= thr).astype(jnp.int32), axis=1, keepdims=True)
        tu = jnp.where(cnt >= _K, cand, tu)
    tkey = tu ^ jnp.int32(_INT_MIN)
    gt = key > tkey
    eq = key == tkey
    need = _K - jnp.sum(gt.astype(jnp.int32), axis=1, keepdims=True)
    r = eq.astype(jnp.int32)
    sh = 1
    while sh < _N:
        r = r + jnp.concatenate(
            [jnp.zeros((_B, sh), jnp.int32), r[:, :-sh]], axis=1)
        sh *= 2
    mask = gt | (eq & (r <= need))
    samp_ref[...] = mask.astype(jnp.float32)


def _run(flat4, pert):
    fpref, bpref = pl.pallas_call(
        _dp_body,
        grid=(_NBLK,),
        in_specs=[
            pl.BlockSpec((1, _R, _B, _G), lambda g: (g, 0, 0, 0)),
            pl.BlockSpec((1, _R, _B, _G), lambda g: (_NBLK - 1 - g, 0, 0, 0)),
        ],
        out_specs=(
            pl.BlockSpec((1, _G, _B, _BLK), lambda g: (g, 0, 0, 0)),
            pl.BlockSpec((1, _G, _B, _BLK), lambda g: (_NBLK - 1 - g, 0, 0, 0)),
        ),
        out_shape=(
            jax.ShapeDtypeStruct((_NBLK, _G, _B, _BLK), jnp.float32),
            jax.ShapeDtypeStruct((_NBLK, _G, _B, _BLK), jnp.float32),
        ),
        scratch_shapes=[
            pltpu.VMEM((_B, _BLK), jnp.float32),
            pltpu.VMEM((_B, _BLK), jnp.float32),
        ],
    )(flat4, flat4)

    l4 = pl.pallas_call(
        _comb_body,
        grid=(_NBLK,),
        in_specs=[
            pl.BlockSpec((1, _R, _B, _G), lambda g: (g, 0, 0, 0)),
            pl.BlockSpec((1, _G, _B, _BLK), lambda g: (g, 0, 0, 0)),
            pl.BlockSpec((1, _G, _B, _BLK), lambda g: (g, 0, 0, 0)),
        ],
        out_specs=pl.BlockSpec((1, _R, _B, _G), lambda g: (g, 0, 0, 0)),
        out_shape=jax.ShapeDtypeStruct((_NBLK, _R, _B, _G), jnp.float32),
    )(flat4, fpref, bpref)

    return pl.pallas_call(
        _fin_body,
        out_shape=(
            jax.ShapeDtypeStruct((_B, _N), jnp.float32),
            jax.ShapeDtypeStruct((_NBLK, _R, _B, _G), jnp.float32),
        ),
    )(l4, pert)


def kernel(scores, times_sampled):
    bsz, nmax, _, ens = scores.shape
    flat = jnp.transpose(scores, (0, 3, 1, 2)).reshape(bsz * ens, nmax * nmax)
    g = jax.random.gumbel(jax.random.key(42), flat.shape, flat.dtype)
    flat4 = flat.reshape(_B, _NBLK, _G, _R).transpose(1, 3, 0, 2)
    samples, marg4 = _run(flat4, flat + g)
    marg = marg4.transpose(2, 0, 3, 1).reshape(_B, _N)
    new_mask = jnp.transpose(
        samples.reshape(1, bsz, ens, nmax, nmax), (0, 1, 3, 4, 2))
    new_marginals = jnp.transpose(
        marg.reshape(bsz, ens, nmax, nmax), (0, 2, 3, 1))
    return new_mask, new_marginals


# speedup vs baseline: 22.5250x; 1.0284x over previous
import jax
import jax.numpy as jnp
from jax.experimental import pallas as pl
from jax.experimental.pallas import tpu as pltpu

_NEG = -1e30
_B = 32
_N = 4096
_K = 128
_BLK = 256
_NBLK = _N // _BLK
_R = 4
_G = _BLK // _R
_INT_MIN = -2147483648


def _lse2(a, b):
    m = jnp.maximum(a, b)
    return m + jnp.log1p(jnp.exp(-jnp.abs(a - b)))


def _logpq(x):
    lp = jnp.minimum(x, 0.0) - jnp.log1p(jnp.exp(-jnp.abs(x)))
    return lp, lp - x


def _group_taps(lp4, lq4):
    z = jnp.zeros_like(lp4[0])
    t = [z, jnp.full_like(z, _NEG), jnp.full_like(z, _NEG),
         jnp.full_like(z, _NEG), jnp.full_like(z, _NEG)]
    for s in range(_R):
        lps, lqs = lp4[s], lq4[s]
        for tau in range(_R, 0, -1):
            t[tau] = _lse2(t[tau] + lqs, t[tau - 1] + lps)
        t[0] = t[0] + lqs
    return t


def _loo_taps(lp4, lq4):
    outs = []
    z = jnp.zeros_like(lp4[0])
    for s in range(_R):
        e = [z, jnp.full_like(z, _NEG), jnp.full_like(z, _NEG),
             jnp.full_like(z, _NEG)]
        for s2 in range(_R):
            if s2 == s:
                continue
            lps, lqs = lp4[s2], lq4[s2]
            for tau in range(_R - 1, 0, -1):
                e[tau] = _lse2(e[tau] + lqs, e[tau - 1] + lps)
            e[0] = e[0] + lqs
        outs.append(e)
    return outs


def _col(tile, j):
    return tile[:, j:j + 1]


def _fused_step(c, taps, reverse):
    neg_col = jnp.full((c.shape[0], 1), _NEG, dtype=c.dtype)
    xs = [c + taps[0]]
    sh = c
    for tau in range(1, _R + 1):
        if reverse:
            sh = jnp.concatenate([sh[:, 1:], neg_col], axis=1)
        else:
            sh = jnp.concatenate([neg_col, sh[:, :-1]], axis=1)
        xs.append(sh + taps[tau])
    m = xs[0]
    for x in xs[1:]:
        m = jnp.maximum(m, x)
    acc = jnp.exp(xs[0] - m)
    for x in xs[1:]:
        acc = acc + jnp.exp(x - m)
    return m + jnp.log(acc)


def _dp_body(flatf_ref, flatb_ref, fout_ref, bout_ref, fc_ref, bc_ref):
    g = pl.program_id(0)
    idsk = jax.lax.broadcasted_iota(jnp.int32, (_B, _K), 1)

    @pl.when(g == 0)
    def _():
        fc_ref[...] = jnp.where(idsk == 0, 0.0, _NEG)
        bc_ref[...] = jnp.where(idsk == _K - 1, 0.0, _NEG)

    lp4f, lq4f = _logpq(flatf_ref[0])
    tf = _group_taps(lp4f, lq4f)
    lp4b, lq4b = _logpq(flatb_ref[0])
    tb = _group_taps(lp4b, lq4b)

    c = fc_ref[...]
    br = bc_ref[...]
    for gg in range(_G):
        fout_ref[0, gg] = c
        bout_ref[0, _G - 1 - gg] = br
        ftaps = [_col(tf[tau], gg) for tau in range(_R + 1)]
        btaps = [_col(tb[tau], _G - 1 - gg) for tau in range(_R + 1)]
        c = _fused_step(c, ftaps, reverse=False)
        br = _fused_step(br, btaps, reverse=True)
    fc_ref[...] = c
    bc_ref[...] = br


def _comb_body(flat4_ref, fpref_ref, bpref_ref, l_ref):
    lp4, lq4 = _logpq(flat4_ref[0])

    fblk = fpref_ref[0]
    brblk = bpref_ref[0]
    dts = []
    for tau in range(_R):
        if tau:
            sh = jnp.concatenate(
                [brblk[:, :, tau:],
                 jnp.full((_G, _B, tau), _NEG, jnp.float32)], axis=2)
        else:
            sh = brblk
        s = fblk + sh
        m = jnp.max(s, axis=2, keepdims=True)
        d = m + jnp.log(jnp.sum(jnp.exp(s - m), axis=2, keepdims=True))
        dts.append(jnp.transpose(d[:, :, 0]))

    loo = _loo_taps(lp4, lq4)
    ls = []
    for s in range(_R):
        xs = [loo[s][tau] + dts[tau] for tau in range(_R)]
        m = xs[0]
        for x in xs[1:]:
            m = jnp.maximum(m, x)
        acc = jnp.exp(xs[0] - m)
        for x in xs[1:]:
            acc = acc + jnp.exp(x - m)
        ls.append(lp4[s] + m + jnp.log(acc))
    l_ref[0] = jnp.stack(ls, axis=0)


def _fin_body(l_ref, pert_ref, samp_ref, marg4_ref):
    l = l_ref[...]
    lm = jnp.max(l, axis=(0, 1, 3), keepdims=True)
    e = jnp.exp(l - lm)
    marg = float(_K) * e / jnp.sum(e, axis=(0, 1, 3), keepdims=True)
    marg4_ref[...] = jnp.clip(marg, 0.0, 1.0)

    bits = jax.lax.bitcast_convert_type(pert_ref[...], jnp.int32)
    key = bits ^ (jnp.int32(0x7FFFFFFF) & (bits >> 31))
    tu = jnp.zeros((_B, 1), jnp.int32)
    for bit in range(31, -1, -1):
        c = 1 << bit
        c = c - (1 << 32) if c >= (1 << 31) else c
        cand = tu | jnp.int32(c)
        thr = cand ^ jnp.int32(_INT_MIN)
        cnt = jnp.sum((key >= thr).astype(jnp.int32), axis=1, keepdims=True)
        tu = jnp.where(cnt >= _K, cand, tu)
    tkey = tu ^ jnp.int32(_INT_MIN)
    gt = key > tkey
    eq = key == tkey
    need = _K - jnp.sum(gt.astype(jnp.int32), axis=1, keepdims=True)
    r = eq.astype(jnp.int32)
    sh = 1
    while sh < _N:
        r = r + jnp.concatenate(
            [jnp.zeros((_B, sh), jnp.int32), r[:, :-sh]], axis=1)
        sh *= 2
    mask = gt | (eq & (r <= need))
    samp_ref[...] = mask.astype(jnp.float32)


def _run(flat4, pert):
    fpref, bpref = pl.pallas_call(
        _dp_body,
        grid=(_NBLK,),
        in_specs=[
            pl.BlockSpec((1, _R, _B, _G), lambda g: (g, 0, 0, 0)),
            pl.BlockSpec((1, _R, _B, _G), lambda g: (_NBLK - 1 - g, 0, 0, 0)),
        ],
        out_specs=(
            pl.BlockSpec((1, _G, _B, _K), lambda g: (g, 0, 0, 0)),
            pl.BlockSpec((1, _G, _B, _K), lambda g: (_NBLK - 1 - g, 0, 0, 0)),
        ),
        out_shape=(
            jax.ShapeDtypeStruct((_NBLK, _G, _B, _K), jnp.float32),
            jax.ShapeDtypeStruct((_NBLK, _G, _B, _K), jnp.float32),
        ),
        scratch_shapes=[
            pltpu.VMEM((_B, _K), jnp.float32),
            pltpu.VMEM((_B, _K), jnp.float32),
        ],
    )(flat4, flat4)

    l4 = pl.pallas_call(
        _comb_body,
        grid=(_NBLK,),
        in_specs=[
            pl.BlockSpec((1, _R, _B, _G), lambda g: (g, 0, 0, 0)),
            pl.BlockSpec((1, _G, _B, _K), lambda g: (g, 0, 0, 0)),
            pl.BlockSpec((1, _G, _B, _K), lambda g: (g, 0, 0, 0)),
        ],
        out_specs=pl.BlockSpec((1, _R, _B, _G), lambda g: (g, 0, 0, 0)),
        out_shape=jax.ShapeDtypeStruct((_NBLK, _R, _B, _G), jnp.float32),
    )(flat4, fpref, bpref)

    return pl.pallas_call(
        _fin_body,
        out_shape=(
            jax.ShapeDtypeStruct((_B, _N), jnp.float32),
            jax.ShapeDtypeStruct((_NBLK, _R, _B, _G), jnp.float32),
        ),
    )(l4, pert)


def kernel(scores, times_sampled):
    bsz, nmax, _, ens = scores.shape
    flat = jnp.transpose(scores, (0, 3, 1, 2)).reshape(bsz * ens, nmax * nmax)
    g = jax.random.gumbel(jax.random.key(42), flat.shape, flat.dtype)
    flat4 = flat.reshape(_B, _NBLK, _G, _R).transpose(1, 3, 0, 2)
    samples, marg4 = _run(flat4, flat + g)
    marg = marg4.transpose(2, 0, 3, 1).reshape(_B, _N)
    new_mask = jnp.transpose(
        samples.reshape(1, bsz, ens, nmax, nmax), (0, 1, 3, 4, 2))
    new_marginals = jnp.transpose(
        marg.reshape(bsz, ens, nmax, nmax), (0, 2, 3, 1))
    return new_mask, new_marginals


# speedup vs baseline: 22.8126x; 1.0128x over previous
import jax
import jax.numpy as jnp
from jax.experimental import pallas as pl
from jax.experimental.pallas import tpu as pltpu

_NEG = -1e30
_B = 32
_N = 4096
_K = 128
_BLK = 512
_NBLK = _N // _BLK
_R = 4
_G = _BLK // _R
_INT_MIN = -2147483648


def _lse2(a, b):
    m = jnp.maximum(a, b)
    return m + jnp.log1p(jnp.exp(-jnp.abs(a - b)))


def _logpq(x):
    lp = jnp.minimum(x, 0.0) - jnp.log1p(jnp.exp(-jnp.abs(x)))
    return lp, lp - x


def _group_taps(lp4, lq4):
    z = jnp.zeros_like(lp4[0])
    t = [z, jnp.full_like(z, _NEG), jnp.full_like(z, _NEG),
         jnp.full_like(z, _NEG), jnp.full_like(z, _NEG)]
    for s in range(_R):
        lps, lqs = lp4[s], lq4[s]
        for tau in range(_R, 0, -1):
            t[tau] = _lse2(t[tau] + lqs, t[tau - 1] + lps)
        t[0] = t[0] + lqs
    return t


def _loo_taps(lp4, lq4):
    outs = []
    z = jnp.zeros_like(lp4[0])
    for s in range(_R):
        e = [z, jnp.full_like(z, _NEG), jnp.full_like(z, _NEG),
             jnp.full_like(z, _NEG)]
        for s2 in range(_R):
            if s2 == s:
                continue
            lps, lqs = lp4[s2], lq4[s2]
            for tau in range(_R - 1, 0, -1):
                e[tau] = _lse2(e[tau] + lqs, e[tau - 1] + lps)
            e[0] = e[0] + lqs
        outs.append(e)
    return outs


def _col(tile, j):
    return tile[:, j:j + 1]


def _fused_step(c, taps, reverse):
    neg_col = jnp.full((c.shape[0], 1), _NEG, dtype=c.dtype)
    xs = [c + taps[0]]
    sh = c
    for tau in range(1, _R + 1):
        if reverse:
            sh = jnp.concatenate([sh[:, 1:], neg_col], axis=1)
        else:
            sh = jnp.concatenate([neg_col, sh[:, :-1]], axis=1)
        xs.append(sh + taps[tau])
    m = xs[0]
    for x in xs[1:]:
        m = jnp.maximum(m, x)
    acc = jnp.exp(xs[0] - m)
    for x in xs[1:]:
        acc = acc + jnp.exp(x - m)
    return m + jnp.log(acc)


def _dp_body(flatf_ref, flatb_ref, fout_ref, bout_ref, fc_ref, bc_ref):
    g = pl.program_id(0)
    idsk = jax.lax.broadcasted_iota(jnp.int32, (_B, _K), 1)

    @pl.when(g == 0)
    def _():
        fc_ref[...] = jnp.where(idsk == 0, 0.0, _NEG)
        bc_ref[...] = jnp.where(idsk == _K - 1, 0.0, _NEG)

    lp4f, lq4f = _logpq(flatf_ref[0])
    tf = _group_taps(lp4f, lq4f)
    lp4b, lq4b = _logpq(flatb_ref[0])
    tb = _group_taps(lp4b, lq4b)

    c = fc_ref[...]
    br = bc_ref[...]
    for gg in range(_G):
        fout_ref[0, gg] = c
        bout_ref[0, _G - 1 - gg] = br
        ftaps = [_col(tf[tau], gg) for tau in range(_R + 1)]
        btaps = [_col(tb[tau], _G - 1 - gg) for tau in range(_R + 1)]
        c = _fused_step(c, ftaps, reverse=False)
        br = _fused_step(br, btaps, reverse=True)
    fc_ref[...] = c
    bc_ref[...] = br


def _comb_body(flat4_ref, fpref_ref, bpref_ref, l_ref):
    lp4, lq4 = _logpq(flat4_ref[0])

    fblk = fpref_ref[0]
    brblk = bpref_ref[0]
    dts = []
    for tau in range(_R):
        if tau:
            sh = jnp.concatenate(
                [brblk[:, :, tau:],
                 jnp.full((_G, _B, tau), _NEG, jnp.float32)], axis=2)
        else:
            sh = brblk
        s = fblk + sh
        m = jnp.max(s, axis=2, keepdims=True)
        d = m + jnp.log(jnp.sum(jnp.exp(s - m), axis=2, keepdims=True))
        dts.append(jnp.transpose(d[:, :, 0]))

    loo = _loo_taps(lp4, lq4)
    ls = []
    for s in range(_R):
        xs = [loo[s][tau] + dts[tau] for tau in range(_R)]
        m = xs[0]
        for x in xs[1:]:
            m = jnp.maximum(m, x)
        acc = jnp.exp(xs[0] - m)
        for x in xs[1:]:
            acc = acc + jnp.exp(x - m)
        ls.append(lp4[s] + m + jnp.log(acc))
    l_ref[0] = jnp.stack(ls, axis=0)


def _fin_body(l_ref, pert_ref, samp_ref, marg4_ref):
    l = l_ref[...]
    lm = jnp.max(l, axis=(0, 1, 3), keepdims=True)
    e = jnp.exp(l - lm)
    marg = float(_K) * e / jnp.sum(e, axis=(0, 1, 3), keepdims=True)
    marg4_ref[...] = jnp.clip(marg, 0.0, 1.0)

    bits = jax.lax.bitcast_convert_type(pert_ref[...], jnp.int32)
    key = bits ^ (jnp.int32(0x7FFFFFFF) & (bits >> 31))
    tu = jnp.zeros((_B, 1), jnp.int32)
    for bit in range(31, -1, -1):
        c = 1 << bit
        c = c - (1 << 32) if c >= (1 << 31) else c
        cand = tu | jnp.int32(c)
        thr = cand ^ jnp.int32(_INT_MIN)
        cnt = jnp.sum((key >= thr).astype(jnp.int32), axis=1, keepdims=True)
        tu = jnp.where(cnt >= _K, cand, tu)
    tkey = tu ^ jnp.int32(_INT_MIN)
    gt = key > tkey
    eq = key == tkey
    need = _K - jnp.sum(gt.astype(jnp.int32), axis=1, keepdims=True)
    r = eq.astype(jnp.int32)
    sh = 1
    while sh < _N:
        r = r + jnp.concatenate(
            [jnp.zeros((_B, sh), jnp.int32), r[:, :-sh]], axis=1)
        sh *= 2
    mask = gt | (eq & (r <= need))
    samp_ref[...] = mask.astype(jnp.float32)


def _run(flat4, pert):
    fpref, bpref = pl.pallas_call(
        _dp_body,
        grid=(_NBLK,),
        in_specs=[
            pl.BlockSpec((1, _R, _B, _G), lambda g: (g, 0, 0, 0)),
            pl.BlockSpec((1, _R, _B, _G), lambda g: (_NBLK - 1 - g, 0, 0, 0)),
        ],
        out_specs=(
            pl.BlockSpec((1, _G, _B, _K), lambda g: (g, 0, 0, 0)),
            pl.BlockSpec((1, _G, _B, _K), lambda g: (_NBLK - 1 - g, 0, 0, 0)),
        ),
        out_shape=(
            jax.ShapeDtypeStruct((_NBLK, _G, _B, _K), jnp.float32),
            jax.ShapeDtypeStruct((_NBLK, _G, _B, _K), jnp.float32),
        ),
        scratch_shapes=[
            pltpu.VMEM((_B, _K), jnp.float32),
            pltpu.VMEM((_B, _K), jnp.float32),
        ],
    )(flat4, flat4)

    l4 = pl.pallas_call(
        _comb_body,
        grid=(_NBLK,),
        in_specs=[
            pl.BlockSpec((1, _R, _B, _G), lambda g: (g, 0, 0, 0)),
            pl.BlockSpec((1, _G, _B, _K), lambda g: (g, 0, 0, 0)),
            pl.BlockSpec((1, _G, _B, _K), lambda g: (g, 0, 0, 0)),
        ],
        out_specs=pl.BlockSpec((1, _R, _B, _G), lambda g: (g, 0, 0, 0)),
        out_shape=jax.ShapeDtypeStruct((_NBLK, _R, _B, _G), jnp.float32),
    )(flat4, fpref, bpref)

    return pl.pallas_call(
        _fin_body,
        out_shape=(
            jax.ShapeDtypeStruct((_B, _N), jnp.float32),
            jax.ShapeDtypeStruct((_NBLK, _R, _B, _G), jnp.float32),
        ),
    )(l4, pert)


def kernel(scores, times_sampled):
    bsz, nmax, _, ens = scores.shape
    flat = jnp.transpose(scores, (0, 3, 1, 2)).reshape(bsz * ens, nmax * nmax)
    g = jax.random.gumbel(jax.random.key(42), flat.shape, flat.dtype)
    flat4 = flat.reshape(_B, _NBLK, _G, _R).transpose(1, 3, 0, 2)
    samples, marg4 = _run(flat4, flat + g)
    marg = marg4.transpose(2, 0, 3, 1).reshape(_B, _N)
    new_mask = jnp.transpose(
        samples.reshape(1, bsz, ens, nmax, nmax), (0, 1, 3, 4, 2))
    new_marginals = jnp.transpose(
        marg.reshape(bsz, ens, nmax, nmax), (0, 2, 3, 1))
    return new_mask, new_marginals
